# baseline (speedup 1.0000x reference)
import functools

import jax
import jax.numpy as jnp
from jax import lax
from jax.experimental import pallas as pl
from jax.experimental.pallas import tpu as pltpu
from jax.experimental.pallas import tpu_sc as plsc

_B = 128
_N = 8192
_NW = 32
_B_SC = 64
_B_TC = _B - _B_SC
_ROWS_PER_T = _B_SC // _NW
_O_PER_T = _ROWS_PER_T * _N
_GROUPS_PER_ROW = _N // 64

_mesh = plsc.VectorSubcoreMesh(core_axis_name="c", subcore_axis_name="s")


@functools.partial(
    pl.kernel,
    mesh=_mesh,
    out_type=jax.ShapeDtypeStruct((_B_SC, _N), jnp.float32),
    scratch_types=[
        pltpu.VMEM((_O_PER_T,), jnp.uint8),
        pltpu.VMEM((_O_PER_T,), jnp.float32),
        pltpu.SemaphoreType.DMA,
        pltpu.SemaphoreType.DMA,
    ],
    compiler_params=pltpu.CompilerParams(needs_layout_passes=False),
)
def _masked_fill_sc(mask_hbm, out_hbm, m_v, o_v, sem_in, sem_out):
    wid = lax.axis_index("s") * 2 + lax.axis_index("c")

    in_handles = [
        pltpu.async_copy(
            mask_hbm.at[pl.ds(wid * _O_PER_T + r * _N, _N)],
            m_v.at[pl.ds(r * _N, _N)],
            sem_in,
        )
        for r in range(_ROWS_PER_T)
    ]

    iota4 = lax.iota(jnp.int32, 16) * 4

    out_handles = []
    for r in range(_ROWS_PER_T):
        in_handles[r].wait()

        def body(g, carry, _r=r):
            base = _r * _N + g * 64
            w = plsc.bitcast(m_v[pl.ds(base, 64)], jnp.uint32)
            for k in range(4):
                if k < 3:
                    word = (w & jnp.uint32(1 << (8 * k))) * jnp.uint32(
                        0xFF800000 >> (8 * k)
                    )
                else:
                    word = (w >> jnp.uint32(24)) * jnp.uint32(0xFF800000)
                plsc.store_scatter(
                    o_v, [iota4 + (base + k)], plsc.bitcast(word, jnp.float32)
                )
            return carry

        lax.fori_loop(0, _GROUPS_PER_ROW, body, 0)
        out_handles.append(
            pltpu.async_copy(
                o_v.at[pl.ds(r * _N, _N)],
                out_hbm.at[wid * _ROWS_PER_T + r],
                sem_out,
            )
        )
    for h in out_handles:
        h.wait()


def _tc_body(mask_ref, out_ref):
    out_ref[...] = jnp.where(
        mask_ref[...], jnp.float32(-jnp.inf), jnp.float32(0.0)
    )


_tc_fill = pl.pallas_call(
    _tc_body,
    out_shape=jax.ShapeDtypeStruct((_B_TC, _N), jnp.float32),
)


def kernel(states, masks):
    del states
    sc_half = _masked_fill_sc(
        masks[_B_TC:].astype(jnp.uint8).reshape(_B_SC * _N)
    )
    tc_half = _tc_fill(masks[:_B_TC])
    return jnp.concatenate([tc_half, sc_half], axis=0)


# speedup vs baseline: 1.1356x; 1.1356x over previous
import functools

import jax
import jax.numpy as jnp
from jax import lax
from jax.experimental import pallas as pl
from jax.experimental.pallas import tpu as pltpu
from jax.experimental.pallas import tpu_sc as plsc

_B = 128
_N = 8192
_NW = 32
_TOTAL = _B * _N
_ROWS_PER_T = _B // _NW
_O_PER_T = _TOTAL // _NW
_GROUPS_PER_ROW = _N // 64

_mesh = plsc.VectorSubcoreMesh(core_axis_name="c", subcore_axis_name="s")


@functools.partial(
    pl.kernel,
    mesh=_mesh,
    out_type=jax.ShapeDtypeStruct((_B, _N), jnp.float32),
    scratch_types=[
        pltpu.VMEM((_O_PER_T,), jnp.uint8),
        pltpu.VMEM((_O_PER_T,), jnp.float32),
        pltpu.SemaphoreType.DMA,
        pltpu.SemaphoreType.DMA,
    ],
    compiler_params=pltpu.CompilerParams(
        needs_layout_passes=False, allow_input_fusion=[True]
    ),
)
def _masked_fill(mask_hbm, out_hbm, m_v, o_v, sem_in, sem_out):
    wid = lax.axis_index("s") * 2 + lax.axis_index("c")

    in_handles = [
        pltpu.async_copy(
            mask_hbm.at[pl.ds(wid * _O_PER_T + r * _N, _N)],
            m_v.at[pl.ds(r * _N, _N)],
            sem_in,
        )
        for r in range(_ROWS_PER_T)
    ]

    iota4 = lax.iota(jnp.int32, 16) * 4
    neg_inf_bits = jnp.uint32(0xFF800000)

    out_handles = []
    for r in range(_ROWS_PER_T):
        in_handles[r].wait()

        def body(g, carry, _r=r):
            base = _r * _N + g * 64
            w = plsc.bitcast(m_v[pl.ds(base, 64)], jnp.uint32)
            for k in range(4):
                byte = (w >> jnp.uint32(8 * k)) & jnp.uint32(1)
                val = plsc.bitcast(byte * neg_inf_bits, jnp.float32)
                plsc.store_scatter(o_v, [iota4 + (base + k)], val)
            return carry

        lax.fori_loop(0, _GROUPS_PER_ROW, body, 0)
        out_handles.append(
            pltpu.async_copy(
                o_v.at[pl.ds(r * _N, _N)],
                out_hbm.at[wid * _ROWS_PER_T + r],
                sem_out,
            )
        )
    for h in out_handles:
        h.wait()


def kernel(states, masks):
    del states
    mask_bytes = masks.astype(jnp.uint8).reshape(_TOTAL)
    return _masked_fill(mask_bytes)
